# baseline (speedup 1.0000x reference)
import jax
import jax.numpy as jnp
from jax.experimental import pallas as pl

_DBLK = 72


def _onehot_t_body(xt_ref, o_ref):
    j, dblk, b = o_ref.shape
    d0 = pl.program_id(0) * dblk
    d = jax.lax.broadcasted_iota(jnp.int32, (j, dblk, b), 1) + d0
    x = xt_ref[...]
    o_ref[...] = (x[:, None, :] == d).astype(jnp.float32)


def kernel(X_in, ones):
    B, J = X_in.shape
    depth = ones.shape[0]
    xt = X_in.T
    t = pl.pallas_call(
        _onehot_t_body,
        grid=(pl.cdiv(depth, _DBLK),),
        in_specs=[pl.BlockSpec((J, B), lambda i: (0, 0))],
        out_specs=pl.BlockSpec((J, _DBLK, B), lambda i: (0, i, 0)),
        out_shape=jax.ShapeDtypeStruct((J, depth, B), jnp.float32),
    )(xt)
    return jnp.transpose(t, (2, 1, 0))


# speedup vs baseline: 1.0240x; 1.0240x over previous
import jax
import jax.numpy as jnp
from jax.experimental import pallas as pl

_DBLK = 56


def _onehot_t_body(xt_ref, o_ref):
    j, dblk, b = o_ref.shape
    d0 = pl.program_id(0) * dblk
    d = jax.lax.broadcasted_iota(jnp.int32, (j, dblk, b), 1) + d0
    x = xt_ref[...]
    o_ref[...] = (x[:, None, :] == d).astype(jnp.float32)


def kernel(X_in, ones):
    B, J = X_in.shape
    depth = ones.shape[0]
    xt = X_in.T
    t = pl.pallas_call(
        _onehot_t_body,
        grid=(pl.cdiv(depth, _DBLK),),
        in_specs=[pl.BlockSpec((J, B), lambda i: (0, 0))],
        out_specs=pl.BlockSpec((J, _DBLK, B), lambda i: (0, i, 0)),
        out_shape=jax.ShapeDtypeStruct((J, depth, B), jnp.float32),
    )(xt)
    return jnp.transpose(t, (2, 1, 0))
